# baseline (speedup 1.0000x reference)
import functools

import jax
import jax.numpy as jnp
from jax import lax
from jax.experimental import pallas as pl
from jax.experimental.pallas import tpu as pltpu
from jax.experimental.pallas import tpu_sc as plsc

N = 10000
NP = 10240
D = 128
E = 320000
NC = 2
NS = 16
NW = NC * NS
RNG = NP // NS
ACC_R = 648
CHUNK = 128
EC = E // NC
SCH = 10
NSUP = EC // (SCH * CHUNK)
NCHMAX = EC // CHUNK
DUMMY = RNG
BLK = 512

_vmesh = plsc.VectorSubcoreMesh(core_axis_name="c", subcore_axis_name="s")


@functools.partial(
    pl.kernel,
    out_type=(
        jax.ShapeDtypeStruct((NW, NCHMAX, 1, CHUNK), jnp.int32),
        jax.ShapeDtypeStruct((NW, NCHMAX, 1, CHUNK), jnp.int32),
        jax.ShapeDtypeStruct((NW, 16), jnp.int32),
        jax.ShapeDtypeStruct((NC * NP,), jnp.float32),
    ),
    mesh=_vmesh,
    compiler_params=pltpu.CompilerParams(needs_layout_passes=False),
    scratch_types=[
        pltpu.VMEM((SCH, CHUNK), jnp.int32),
        pltpu.VMEM((SCH, CHUNK), jnp.int32),
        pltpu.VMEM((256,), jnp.int32),
        pltpu.VMEM((256,), jnp.int32),
        pltpu.VMEM((16,), jnp.int32),
        pltpu.VMEM((656,), jnp.float32),
    ],
)
def _sc_bucket(row_hbm, col_hbm,
               rlist, clist, counts, degw,
               ebr, ebc, rbuf, cbuf, cntv, dacc):
    cid = lax.axis_index("c")
    sid = lax.axis_index("s")
    wid = cid * NS + sid

    zf = jnp.zeros((16,), jnp.float32)

    def zrow(i, carry):
        dacc[pl.ds(i * 16, 16)] = zf
        return carry

    lax.fori_loop(0, 656 // 16, zrow, 0)

    lo = (sid * RNG).astype(jnp.int32)
    fone = jnp.ones((16,), jnp.float32)

    def flush(off, nfl):
        pltpu.sync_copy(rbuf.at[pl.ds(0, CHUNK)], rlist.at[wid, nfl, 0])
        pltpu.sync_copy(cbuf.at[pl.ds(0, CHUNK)], clist.at[wid, nfl, 0])
        for g in range(CHUNK // 16):
            cl16 = cbuf[pl.ds(g * 16, 16)]
            plsc.addupdate_scatter(dacc, [cl16], fone)
        rbuf[pl.ds(0, 16)] = rbuf[pl.ds(CHUNK, 16)]
        cbuf[pl.ds(0, 16)] = cbuf[pl.ds(CHUNK, 16)]

    def scan_group(ebr_ref, ebc_ref, i, j, off, nfl):
        r16 = ebr_ref[i, pl.ds(j * 16, 16)]
        c16 = ebc_ref[i, pl.ds(j * 16, 16)]
        msk = (c16 >= lo) & (c16 < lo + RNG)
        cl16 = c16 - lo
        mi = jnp.where(msk, jnp.int32(1), jnp.int32(0))
        pos = plsc.cumsum(mi)
        idx = jnp.where(msk, off + pos - mi, jnp.int32(255))
        plsc.store_scatter(rbuf, [idx], r16)
        plsc.store_scatter(cbuf, [idx], cl16)
        n = pos[15]
        off2 = off + n
        do = off2 >= CHUNK

        @pl.when(do)
        def _():
            flush(off2, nfl)

        return lax.select(do, off2 - CHUNK, off2), nfl + do.astype(jnp.int32)

    def super_step(g, carry):
        off, nfl = carry
        pltpu.sync_copy(row_hbm.at[cid, g], ebr)
        pltpu.sync_copy(col_hbm.at[cid, g], ebc)

        def group(k, carry2):
            off2, nfl2 = carry2
            return scan_group(ebr, ebc, k // 8, lax.rem(k, 8), off2, nfl2)

        return lax.fori_loop(0, SCH * 8, group, (off, nfl))

    off, nfl = lax.fori_loop(0, NSUP, super_step,
                             (jnp.int32(0), jnp.int32(0)))

    zv16 = jnp.zeros((16,), jnp.int32)
    dv16 = jnp.full((16,), DUMMY, jnp.int32)
    for k in range(8):
        @pl.when(off + k * 16 < CHUNK)
        def _():
            rbuf[pl.ds(off + k * 16, 16)] = zv16
            cbuf[pl.ds(off + k * 16, 16)] = dv16
    flush(jnp.int32(CHUNK), nfl)

    cnt = nfl * CHUNK + off
    cntv[:] = jnp.full((16,), 1, jnp.int32) * cnt
    pltpu.sync_copy(cntv, counts.at[wid])

    pltpu.sync_copy(dacc.at[pl.ds(0, RNG)],
                    degw.at[pl.ds(cid * NP + sid * RNG, RNG)])


@functools.partial(
    pl.kernel,
    out_type=jax.ShapeDtypeStruct((NC * NP, D), jnp.float32),
    mesh=_vmesh,
    compiler_params=pltpu.CompilerParams(needs_layout_passes=False),
    scratch_types=[
        pltpu.VMEM((16,), jnp.int32),
        pltpu.VMEM((1, CHUNK), jnp.int32),
        pltpu.VMEM((1, CHUNK), jnp.int32),
        pltpu.VMEM((CHUNK, D), jnp.float32),
        pltpu.VMEM((ACC_R, D), jnp.float32),
        pltpu.SemaphoreType.DMA,
    ],
)
def _sc_scatter(m_hbm, rlist, clist, counts, out_hbm,
                cnt_s, rv, cv, rows_v, acc, gsem):
    cid = lax.axis_index("c")
    sid = lax.axis_index("s")
    wid = cid * NS + sid
    pltpu.sync_copy(counts.at[wid], cnt_s)

    zf = jnp.zeros((16,), jnp.float32)

    @pl.when(cid == 0)
    def _():
        pltpu.sync_copy(m_hbm.at[pl.ds(sid * RNG, RNG)], acc.at[pl.ds(0, RNG)])

        def zdrow(i, carry):
            for j in range(D // 16):
                acc[RNG + i, pl.ds(j * 16, 16)] = zf
            return carry

        lax.fori_loop(0, ACC_R - RNG, zdrow, 0)

    @pl.when(cid == 1)
    def _():
        def zarow(i, carry):
            for j in range(D // 16):
                acc[i, pl.ds(j * 16, 16)] = zf
            return carry

        lax.fori_loop(0, ACC_R, zarow, 0)

    cnt = lax.shift_right_logical(jnp.sum(cnt_s[:]), 4)
    nch = lax.shift_right_logical(cnt + (CHUNK - 1), 7)
    lanes = lax.iota(jnp.int32, 16)

    def chunk(j, carry):
        pltpu.sync_copy(rlist.at[wid, j], rv)
        pltpu.sync_copy(clist.at[wid, j], cv)
        pltpu.async_copy(m_hbm.at[rv.at[0]], rows_v, gsem).wait()

        def group(g, carry2):
            cl16 = cv[0, pl.ds(g * 16, 16)]
            e16 = lanes + g * 16
            for c in range(D):
                c16 = jnp.full((16,), c, jnp.int32)
                v = plsc.load_gather(rows_v, [e16, c16])
                plsc.addupdate_scatter(acc, [cl16, c16], v)
            return carry2

        lax.fori_loop(0, CHUNK // 16, group, 0)
        return carry

    lax.fori_loop(0, nch, chunk, 0)

    pltpu.sync_copy(acc.at[pl.ds(0, RNG)],
                    out_hbm.at[pl.ds(cid * NP + sid * RNG, RNG)])


def _z1_body(x_ref, w_ref, o_ref):
    o_ref[:, :] = jnp.dot(x_ref[:, :], w_ref[:, :],
                          preferred_element_type=jnp.float32)


_z1 = pl.pallas_call(
    _z1_body,
    grid=(NP // BLK,),
    in_specs=[pl.BlockSpec((BLK, D), lambda i: (i, 0)),
              pl.BlockSpec((D, D), lambda i: (0, 0))],
    out_specs=pl.BlockSpec((BLK, D), lambda i: (i, 0)),
    out_shape=jax.ShapeDtypeStruct((NP, D), jnp.float32),
)


def _m1_body(z_ref, degw_ref, o_ref):
    deg = degw_ref[0, :] + degw_ref[1, :] + 1.0
    d = lax.rsqrt(deg)
    o_ref[:, :] = z_ref[:, :] * d[:, None]


_m1 = pl.pallas_call(
    _m1_body,
    grid=(NP // BLK,),
    in_specs=[pl.BlockSpec((BLK, D), lambda i: (i, 0)),
              pl.BlockSpec((NC, BLK), lambda i: (0, i))],
    out_specs=pl.BlockSpec((BLK, D), lambda i: (i, 0)),
    out_shape=jax.ShapeDtypeStruct((NP, D), jnp.float32),
)


def _l2_body(p_ref, degw_ref, b_ref, w_ref, o_ref):
    deg = degw_ref[0, :] + degw_ref[1, :] + 1.0
    d = lax.rsqrt(deg)
    s = p_ref[0] + p_ref[1]
    a = jnp.maximum(s * d[:, None] + b_ref[0, :][None, :], 0.0)
    z = jnp.dot(a, w_ref[:, :], preferred_element_type=jnp.float32)
    o_ref[:, :] = z * d[:, None]


_l2 = pl.pallas_call(
    _l2_body,
    grid=(NP // BLK,),
    in_specs=[pl.BlockSpec((NC, BLK, D), lambda i: (0, i, 0)),
              pl.BlockSpec((NC, BLK), lambda i: (0, i)),
              pl.BlockSpec((1, D), lambda i: (0, 0)),
              pl.BlockSpec((D, D), lambda i: (0, 0))],
    out_specs=pl.BlockSpec((BLK, D), lambda i: (i, 0)),
    out_shape=jax.ShapeDtypeStruct((NP, D), jnp.float32),
)


def _l3_body(p_ref, degw_ref, b_ref, wf_ref, bf_ref, o_ref):
    deg = degw_ref[0, :] + degw_ref[1, :] + 1.0
    d = lax.rsqrt(deg)
    s = p_ref[0] + p_ref[1]
    a = jnp.maximum(s * d[:, None] + b_ref[0, :][None, :], 0.0)
    y = jnp.dot(a, wf_ref[:, :], preferred_element_type=jnp.float32)
    o_ref[:, :] = y + bf_ref[0, :][None, :]


_l3 = pl.pallas_call(
    _l3_body,
    grid=(NP // BLK,),
    in_specs=[pl.BlockSpec((NC, BLK, D), lambda i: (0, i, 0)),
              pl.BlockSpec((NC, BLK), lambda i: (0, i)),
              pl.BlockSpec((1, D), lambda i: (0, 0)),
              pl.BlockSpec((D, D), lambda i: (0, 0)),
              pl.BlockSpec((1, D), lambda i: (0, 0))],
    out_specs=pl.BlockSpec((BLK, D), lambda i: (i, 0)),
    out_shape=jax.ShapeDtypeStruct((NP, D), jnp.float32),
)


def kernel(x, edge_index, W1, b1, W2, b2, Wf, bf):
    x_pad = jnp.pad(x, ((0, NP - N), (0, 0)))
    row = edge_index[0].reshape(NC, NSUP, SCH, CHUNK)
    col = edge_index[1].reshape(NC, NSUP, SCH, CHUNK)

    rlist, clist, counts, degw_f = _sc_bucket(row, col)
    degw = degw_f.reshape(NC, NP)
    z1 = _z1(x_pad, W1)
    m1 = _m1(z1, degw)
    p1 = _sc_scatter(m1, rlist, clist, counts).reshape(NC, NP, D)
    m2 = _l2(p1, degw, b1.reshape(1, D), W2)
    p2 = _sc_scatter(m2, rlist, clist, counts).reshape(NC, NP, D)
    wf_pad = jnp.pad(Wf, ((0, 0), (0, D - 1)))
    bf_pad = jnp.pad(bf.reshape(1, 1), ((0, 0), (0, D - 1)))
    y = _l3(p2, degw, b2.reshape(1, D), wf_pad, bf_pad)
    return y[:N, :1]


# speedup vs baseline: 1.0521x; 1.0521x over previous
import functools

import jax
import jax.numpy as jnp
from jax import lax
from jax.experimental import pallas as pl
from jax.experimental.pallas import tpu as pltpu
from jax.experimental.pallas import tpu_sc as plsc

N = 10000
NP = 10240
D = 128
E = 320000
NC = 2
NS = 16
NW = NC * NS
RNG = NP // NS
ACC_R = 648
CHUNK = 128
EC = E // NC
SCH = 10
NSUP = EC // (SCH * CHUNK)
NCHMAX = EC // CHUNK
DUMMY = RNG
BLK = 512

_vmesh = plsc.VectorSubcoreMesh(core_axis_name="c", subcore_axis_name="s")


@functools.partial(
    pl.kernel,
    out_type=(
        jax.ShapeDtypeStruct((NW, NCHMAX, 1, CHUNK), jnp.int32),
        jax.ShapeDtypeStruct((NW, NCHMAX, 1, CHUNK), jnp.int32),
        jax.ShapeDtypeStruct((NW, 16), jnp.int32),
        jax.ShapeDtypeStruct((NC * NP,), jnp.float32),
    ),
    mesh=_vmesh,
    compiler_params=pltpu.CompilerParams(needs_layout_passes=False),
    scratch_types=[
        pltpu.VMEM((SCH, CHUNK), jnp.int32),
        pltpu.VMEM((SCH, CHUNK), jnp.int32),
        pltpu.VMEM((256,), jnp.int32),
        pltpu.VMEM((256,), jnp.int32),
        pltpu.VMEM((16,), jnp.int32),
        pltpu.VMEM((656,), jnp.float32),
    ],
)
def _sc_bucket(row_hbm, col_hbm,
               rlist, clist, counts, degw,
               ebr, ebc, rbuf, cbuf, cntv, dacc):
    cid = lax.axis_index("c")
    sid = lax.axis_index("s")
    wid = cid * NS + sid

    zf = jnp.zeros((16,), jnp.float32)

    def zrow(i, carry):
        dacc[pl.ds(i * 16, 16)] = zf
        return carry

    lax.fori_loop(0, 656 // 16, zrow, 0)

    lo = (sid * RNG).astype(jnp.int32)
    fone = jnp.ones((16,), jnp.float32)

    def flush(off, nfl):
        pltpu.sync_copy(rbuf.at[pl.ds(0, CHUNK)], rlist.at[wid, nfl, 0])
        pltpu.sync_copy(cbuf.at[pl.ds(0, CHUNK)], clist.at[wid, nfl, 0])
        for g in range(CHUNK // 16):
            cl16 = cbuf[pl.ds(g * 16, 16)]
            plsc.addupdate_scatter(dacc, [cl16], fone)
        rbuf[pl.ds(0, 16)] = rbuf[pl.ds(CHUNK, 16)]
        cbuf[pl.ds(0, 16)] = cbuf[pl.ds(CHUNK, 16)]

    def scan_group(ebr_ref, ebc_ref, i, j, off, nfl):
        r16 = ebr_ref[i, pl.ds(j * 16, 16)]
        c16 = ebc_ref[i, pl.ds(j * 16, 16)]
        msk = (c16 >= lo) & (c16 < lo + RNG)
        cl16 = c16 - lo
        mi = jnp.where(msk, jnp.int32(1), jnp.int32(0))
        pos = plsc.cumsum(mi)
        idx = jnp.where(msk, off + pos - mi, jnp.int32(255))
        plsc.store_scatter(rbuf, [idx], r16)
        plsc.store_scatter(cbuf, [idx], cl16)
        n = pos[15]
        off2 = off + n
        do = off2 >= CHUNK

        @pl.when(do)
        def _():
            flush(off2, nfl)

        return lax.select(do, off2 - CHUNK, off2), nfl + do.astype(jnp.int32)

    def super_step(g, carry):
        off, nfl = carry
        pltpu.sync_copy(row_hbm.at[cid, g], ebr)
        pltpu.sync_copy(col_hbm.at[cid, g], ebc)

        def group(k, carry2):
            off2, nfl2 = carry2
            return scan_group(ebr, ebc, k // 8, lax.rem(k, 8), off2, nfl2)

        return lax.fori_loop(0, SCH * 8, group, (off, nfl))

    off, nfl = lax.fori_loop(0, NSUP, super_step,
                             (jnp.int32(0), jnp.int32(0)))

    zv16 = jnp.zeros((16,), jnp.int32)
    dv16 = jnp.full((16,), DUMMY, jnp.int32)
    for k in range(8):
        @pl.when(off + k * 16 < CHUNK)
        def _():
            rbuf[pl.ds(off + k * 16, 16)] = zv16
            cbuf[pl.ds(off + k * 16, 16)] = dv16
    flush(jnp.int32(CHUNK), nfl)

    cnt = nfl * CHUNK + off
    cntv[:] = jnp.full((16,), 1, jnp.int32) * cnt
    pltpu.sync_copy(cntv, counts.at[wid])

    pltpu.sync_copy(dacc.at[pl.ds(0, RNG)],
                    degw.at[pl.ds(cid * NP + sid * RNG, RNG)])


@functools.partial(
    pl.kernel,
    out_type=jax.ShapeDtypeStruct((NC * NP, D), jnp.float32),
    mesh=_vmesh,
    compiler_params=pltpu.CompilerParams(needs_layout_passes=False),
    scratch_types=[
        pltpu.VMEM((16,), jnp.int32),
        pltpu.VMEM((2, 1, CHUNK), jnp.int32),
        pltpu.VMEM((2, 1, CHUNK), jnp.int32),
        pltpu.VMEM((2, CHUNK, D), jnp.float32),
        pltpu.VMEM((ACC_R, D), jnp.float32),
        pltpu.SemaphoreType.DMA,
    ],
)
def _sc_scatter(m_hbm, rlist, clist, counts, out_hbm,
                cnt_s, rv, cv, rows_v, acc, gsem):
    cid = lax.axis_index("c")
    sid = lax.axis_index("s")
    wid = cid * NS + sid
    pltpu.sync_copy(counts.at[wid], cnt_s)

    zf = jnp.zeros((16,), jnp.float32)

    @pl.when(cid == 0)
    def _():
        pltpu.sync_copy(m_hbm.at[pl.ds(sid * RNG, RNG)], acc.at[pl.ds(0, RNG)])

        def zdrow(i, carry):
            for j in range(D // 16):
                acc[RNG + i, pl.ds(j * 16, 16)] = zf
            return carry

        lax.fori_loop(0, ACC_R - RNG, zdrow, 0)

    @pl.when(cid == 1)
    def _():
        def zarow(i, carry):
            for j in range(D // 16):
                acc[i, pl.ds(j * 16, 16)] = zf
            return carry

        lax.fori_loop(0, ACC_R, zarow, 0)

    cnt = lax.shift_right_logical(jnp.sum(cnt_s[:]), 4)
    nch = lax.shift_right_logical(cnt + (CHUNK - 1), 7)
    lanes = lax.iota(jnp.int32, 16)

    @pl.when(nch > 0)
    def _():
        pltpu.sync_copy(rlist.at[wid, 0], rv.at[0])
        pltpu.sync_copy(clist.at[wid, 0], cv.at[0])
        pltpu.async_copy(m_hbm.at[rv.at[0, 0]], rows_v.at[0], gsem)

        def chunk(j, carry):
            p = lax.rem(j, 2)
            pn = lax.rem(j + 1, 2)

            @pl.when(j + 1 < nch)
            def _():
                pltpu.sync_copy(rlist.at[wid, j + 1], rv.at[pn])
                pltpu.sync_copy(clist.at[wid, j + 1], cv.at[pn])

            pltpu.make_async_copy(m_hbm.at[rv.at[p, 0]], rows_v.at[p],
                                  gsem).wait()

            @pl.when(j + 1 < nch)
            def _():
                pltpu.async_copy(m_hbm.at[rv.at[pn, 0]], rows_v.at[pn], gsem)

            def group(g, carry2):
                cl16 = cv[p, 0, pl.ds(g * 16, 16)]
                e16 = lanes + g * 16
                for c in range(D):
                    c16 = jnp.full((16,), c, jnp.int32)
                    v = plsc.load_gather(rows_v.at[p], [e16, c16])
                    plsc.addupdate_scatter(acc, [cl16, c16], v)
                return carry2

            lax.fori_loop(0, CHUNK // 16, group, 0)
            return carry

        lax.fori_loop(0, nch, chunk, 0)

    pltpu.sync_copy(acc.at[pl.ds(0, RNG)],
                    out_hbm.at[pl.ds(cid * NP + sid * RNG, RNG)])


def _z1_body(x_ref, w_ref, o_ref):
    o_ref[:, :] = jnp.dot(x_ref[:, :], w_ref[:, :],
                          preferred_element_type=jnp.float32)


_z1 = pl.pallas_call(
    _z1_body,
    grid=(NP // BLK,),
    in_specs=[pl.BlockSpec((BLK, D), lambda i: (i, 0)),
              pl.BlockSpec((D, D), lambda i: (0, 0))],
    out_specs=pl.BlockSpec((BLK, D), lambda i: (i, 0)),
    out_shape=jax.ShapeDtypeStruct((NP, D), jnp.float32),
)


def _m1_body(z_ref, degw_ref, o_ref):
    deg = degw_ref[0, :] + degw_ref[1, :] + 1.0
    d = lax.rsqrt(deg)
    o_ref[:, :] = z_ref[:, :] * d[:, None]


_m1 = pl.pallas_call(
    _m1_body,
    grid=(NP // BLK,),
    in_specs=[pl.BlockSpec((BLK, D), lambda i: (i, 0)),
              pl.BlockSpec((NC, BLK), lambda i: (0, i))],
    out_specs=pl.BlockSpec((BLK, D), lambda i: (i, 0)),
    out_shape=jax.ShapeDtypeStruct((NP, D), jnp.float32),
)


def _l2_body(p_ref, degw_ref, b_ref, w_ref, o_ref):
    deg = degw_ref[0, :] + degw_ref[1, :] + 1.0
    d = lax.rsqrt(deg)
    s = p_ref[0] + p_ref[1]
    a = jnp.maximum(s * d[:, None] + b_ref[0, :][None, :], 0.0)
    z = jnp.dot(a, w_ref[:, :], preferred_element_type=jnp.float32)
    o_ref[:, :] = z * d[:, None]


_l2 = pl.pallas_call(
    _l2_body,
    grid=(NP // BLK,),
    in_specs=[pl.BlockSpec((NC, BLK, D), lambda i: (0, i, 0)),
              pl.BlockSpec((NC, BLK), lambda i: (0, i)),
              pl.BlockSpec((1, D), lambda i: (0, 0)),
              pl.BlockSpec((D, D), lambda i: (0, 0))],
    out_specs=pl.BlockSpec((BLK, D), lambda i: (i, 0)),
    out_shape=jax.ShapeDtypeStruct((NP, D), jnp.float32),
)


def _l3_body(p_ref, degw_ref, b_ref, wf_ref, bf_ref, o_ref):
    deg = degw_ref[0, :] + degw_ref[1, :] + 1.0
    d = lax.rsqrt(deg)
    s = p_ref[0] + p_ref[1]
    a = jnp.maximum(s * d[:, None] + b_ref[0, :][None, :], 0.0)
    y = jnp.dot(a, wf_ref[:, :], preferred_element_type=jnp.float32)
    o_ref[:, :] = y + bf_ref[0, :][None, :]


_l3 = pl.pallas_call(
    _l3_body,
    grid=(NP // BLK,),
    in_specs=[pl.BlockSpec((NC, BLK, D), lambda i: (0, i, 0)),
              pl.BlockSpec((NC, BLK), lambda i: (0, i)),
              pl.BlockSpec((1, D), lambda i: (0, 0)),
              pl.BlockSpec((D, D), lambda i: (0, 0)),
              pl.BlockSpec((1, D), lambda i: (0, 0))],
    out_specs=pl.BlockSpec((BLK, D), lambda i: (i, 0)),
    out_shape=jax.ShapeDtypeStruct((NP, D), jnp.float32),
)


def kernel(x, edge_index, W1, b1, W2, b2, Wf, bf):
    x_pad = jnp.pad(x, ((0, NP - N), (0, 0)))
    row = edge_index[0].reshape(NC, NSUP, SCH, CHUNK)
    col = edge_index[1].reshape(NC, NSUP, SCH, CHUNK)

    rlist, clist, counts, degw_f = _sc_bucket(row, col)
    degw = degw_f.reshape(NC, NP)
    z1 = _z1(x_pad, W1)
    m1 = _m1(z1, degw)
    p1 = _sc_scatter(m1, rlist, clist, counts).reshape(NC, NP, D)
    m2 = _l2(p1, degw, b1.reshape(1, D), W2)
    p2 = _sc_scatter(m2, rlist, clist, counts).reshape(NC, NP, D)
    wf_pad = jnp.pad(Wf, ((0, 0), (0, D - 1)))
    bf_pad = jnp.pad(bf.reshape(1, 1), ((0, 0), (0, D - 1)))
    y = _l3(p2, degw, b2.reshape(1, D), wf_pad, bf_pad)
    return y[:N, :1]
